# baseline (speedup 1.0000x reference)
import functools

import jax
import jax.numpy as jnp
from jax import lax
from jax.experimental import pallas as pl
from jax.experimental.pallas import tpu as pltpu
from jax.experimental.pallas import tpu_sc as plsc

T = 16384
E = 64
D = 64
ROWS = T * E
NC, NS, L = 2, 16, 16
NW = NC * NS
ROWS_W = ROWS // NW
CHUNK = 128
NCHUNK = ROWS_W // CHUNK


def _unpermute_body(x_hbm, out_hbm, idx_v, buf_v, gsem):
    wid = lax.axis_index("s") * NC + lax.axis_index("c")
    rbase0 = wid * ROWS_W
    lanes = lax.iota(jnp.int32, L)

    def chunk_body(ci, carry):
        rbase = rbase0 + ci * CHUNK
        for s in range(CHUNK // L):
            base = rbase + E * ((s * L) // E) + (E - 1) - (s * L) % E
            idx_v[pl.ds(s * L, L)] = base - lanes
        pltpu.async_copy(x_hbm.at[idx_v], buf_v, gsem).wait()
        pltpu.sync_copy(buf_v, out_hbm.at[pl.ds(rbase, CHUNK)])
        return carry

    lax.fori_loop(0, NCHUNK, chunk_body, 0)


def kernel(x):
    x2 = x.reshape(ROWS, D)
    mesh = plsc.VectorSubcoreMesh(core_axis_name="c", subcore_axis_name="s")
    run = functools.partial(
        pl.kernel,
        mesh=mesh,
        out_type=jax.ShapeDtypeStruct((ROWS, D), jnp.float32),
        scratch_types=[
            pltpu.VMEM((CHUNK,), jnp.int32),
            pltpu.VMEM((CHUNK, D), jnp.float32),
            pltpu.SemaphoreType.DMA,
        ],
        compiler_params=pltpu.CompilerParams(use_tc_tiling_on_sc=False),
    )(_unpermute_body)
    y2 = run(x2)
    return y2.reshape(T, E, D)


# speedup vs baseline: 1.1338x; 1.1338x over previous
import functools

import jax
import jax.numpy as jnp
from jax import lax
from jax.experimental import pallas as pl
from jax.experimental.pallas import tpu as pltpu
from jax.experimental.pallas import tpu_sc as plsc

T = 16384
E = 64
D = 64
ROWS = T * E
NC, NS, L = 2, 16, 16
NW = NC * NS
ROWS_W = ROWS // NW
CHUNK = 128
NCHUNK = ROWS_W // CHUNK
NBUF = 4


def _unpermute_body(x_hbm, out_hbm, idx_v, buf_v, *sems):
    gsem, wsem = sems[:NBUF], sems[NBUF:]
    wid = lax.axis_index("s") * NC + lax.axis_index("c")
    rbase0 = wid * ROWS_W
    lanes = lax.iota(jnp.int32, L)

    def build_idx(b, ci):
        rbase = rbase0 + ci * CHUNK
        for s in range(CHUNK // L):
            base = rbase + E * ((s * L) // E) + (E - 1) - (s * L) % E
            idx_v[b, pl.ds(s * L, L)] = base - lanes

    def gather(b):
        return pltpu.make_async_copy(x_hbm.at[idx_v.at[b]], buf_v.at[b], gsem[b])

    def writeback(b, ci):
        rbase = rbase0 + ci * CHUNK
        return pltpu.make_async_copy(
            buf_v.at[b], out_hbm.at[pl.ds(rbase, CHUNK)], wsem[b])

    for b in range(NBUF):
        build_idx(b, b)
        gather(b).start()

    def group_body(g, carry):
        for b in range(NBUF):
            ci = g * NBUF + b
            gather(b).wait()
            writeback(b, ci).start()
            writeback(b, ci).wait()
            build_idx(b, ci + NBUF)
            gather(b).start()
        return carry

    lax.fori_loop(0, NCHUNK // NBUF - 1, group_body, 0)

    g_last = NCHUNK // NBUF - 1
    for b in range(NBUF):
        ci = g_last * NBUF + b
        gather(b).wait()
        writeback(b, ci).start()
    for b in range(NBUF):
        writeback(b, g_last * NBUF + b).wait()


def kernel(x):
    x2 = x.reshape(ROWS, D)
    mesh = plsc.VectorSubcoreMesh(core_axis_name="c", subcore_axis_name="s")
    run = functools.partial(
        pl.kernel,
        mesh=mesh,
        out_type=jax.ShapeDtypeStruct((ROWS, D), jnp.float32),
        scratch_types=[
            pltpu.VMEM((NBUF, CHUNK), jnp.int32),
            pltpu.VMEM((NBUF, CHUNK, D), jnp.float32),
        ] + [pltpu.SemaphoreType.DMA] * (2 * NBUF),
        compiler_params=pltpu.CompilerParams(use_tc_tiling_on_sc=False),
    )(_unpermute_body)
    y2 = run(x2)
    return y2.reshape(T, E, D)
